# baseline (speedup 1.0000x reference)
import math

import jax
import jax.numpy as jnp
from jax.experimental import pallas as pl
from jax.experimental.pallas import tpu as pltpu

TK = 32


def _body(eps_ref, a_ref, x_ref, w1_ref, b1_ref, g1_ref, be1_ref,
          w2_ref, b2_ref, g2_ref, be2_ref, v1_ref, c1_ref, v2_ref, c2_ref,
          out_ref, q_ref, g_ref):
    n = a_ref.shape[1]
    d = x_ref.shape[2]
    nt = n // TK
    f32 = jnp.float32
    a = a_ref[0]
    x = x_ref[0]
    g_ref[:, :] = jnp.dot(x, v1_ref[:, :], preferred_element_type=f32) \
        + 0.5 * c1_ref[:, :]
    G = g_ref[:, :]
    c2 = c2_ref[:, :]
    v2 = v2_ref[:, :]
    rowi = jax.lax.broadcasted_iota(jnp.int32, (n, n), 0)
    coli = jax.lax.broadcasted_iota(jnp.int32, (n, n), 1)
    atl = jnp.where(rowi > coli, a, 0.0)
    c2t = c2.reshape(-1, 1)
    for t in range(nt):
        rmax = (t + 1) * TK
        gk = G[t * TK:(t + 1) * TK, :]
        h = gk[:, None, :] + G[:rmax][None, :, :]
        h = jnp.maximum(h, 0.0).reshape(TK * rmax, -1)
        pt = jax.lax.dot_general(
            v2, h, dimension_numbers=(((0,), (1,)), ((), ())),
            preferred_element_type=f32) + c2t
        pt = jnp.maximum(pt, 0.0)
        for kk in range(TK):
            k = t * TK + kk
            q_ref[k * d:(k + 1) * d, :rmax] = (
                atl[k:k + 1, :rmax] * pt[:, kk * rmax:(kk + 1) * rmax])
    accT = jnp.zeros((d, n), dtype=f32)
    for t in range(nt):
        rmax = (t + 1) * TK
        yt = jax.lax.dot_general(
            q_ref[t * TK * d:(t + 1) * TK * d, :rmax], a[:, :rmax],
            dimension_numbers=(((1,), (1,)), ((), ())),
            preferred_element_type=f32)
        for kk in range(TK):
            k = t * TK + kk
            accT = accT + a[k:k + 1, :] * yt[kk * d:(kk + 1) * d, :]
    acc = accT.T
    xc = (1.0 + eps_ref[0, 0]) * x + jnp.dot(a, x, preferred_element_type=f32) \
        + acc
    inv = 1.0 / math.sqrt(1.0 + 1e-5)
    h1 = jnp.maximum(
        jnp.dot(xc, w1_ref[:, :], preferred_element_type=f32) + b1_ref[:, :], 0.0)
    h1 = h1 * (inv * g1_ref[:, :]) + be1_ref[:, :]
    h2 = jnp.maximum(
        jnp.dot(h1, w2_ref[:, :], preferred_element_type=f32) + b2_ref[:, :], 0.0)
    out_ref[0] = h2 * (inv * g2_ref[:, :]) + be2_ref[:, :]


def kernel(A, X, eps, W1, b1, g1, be1, W2, b2, g2, be2, V1, c1, V2, c2):
    b, n = A.shape[0], A.shape[1]
    d_in, d_h = W1.shape
    fixed = lambda *zeros: (lambda i: zeros)
    out = pl.pallas_call(
        _body,
        grid=(b,),
        in_specs=[
            pl.BlockSpec((1, 1), fixed(0, 0), memory_space=pltpu.SMEM),
            pl.BlockSpec((1, n, n), lambda i: (i, 0, 0)),
            pl.BlockSpec((1, n, d_in), lambda i: (i, 0, 0)),
            pl.BlockSpec((d_in, d_h), fixed(0, 0)),
            pl.BlockSpec((1, d_h), fixed(0, 0)),
            pl.BlockSpec((n, 1), fixed(0, 0)),
            pl.BlockSpec((n, 1), fixed(0, 0)),
            pl.BlockSpec((d_h, d_h), fixed(0, 0)),
            pl.BlockSpec((1, d_h), fixed(0, 0)),
            pl.BlockSpec((n, 1), fixed(0, 0)),
            pl.BlockSpec((n, 1), fixed(0, 0)),
            pl.BlockSpec((d_in, d_h), fixed(0, 0)),
            pl.BlockSpec((1, d_h), fixed(0, 0)),
            pl.BlockSpec((d_h, d_in), fixed(0, 0)),
            pl.BlockSpec((1, d_in), fixed(0, 0)),
        ],
        out_specs=pl.BlockSpec((1, n, d_h), lambda i: (i, 0, 0)),
        out_shape=jax.ShapeDtypeStruct((b, n, d_h), jnp.float32),
        scratch_shapes=[
            pltpu.VMEM((n * d_in, n), jnp.float32),
            pltpu.VMEM((n, d_h), jnp.float32),
        ],
        compiler_params=pltpu.CompilerParams(
            dimension_semantics=("parallel",),
        ),
    )(
        eps.reshape(1, 1), A, X, W1, b1.reshape(1, d_h), g1.reshape(n, 1),
        be1.reshape(n, 1), W2, b2.reshape(1, d_h), g2.reshape(n, 1),
        be2.reshape(n, 1), V1, c1.reshape(1, d_h), V2, c2.reshape(1, d_in),
    )
    return out


# speedup vs baseline: 1.0696x; 1.0696x over previous
import math

import jax
import jax.numpy as jnp
from jax.experimental import pallas as pl
from jax.experimental.pallas import tpu as pltpu

TK = 16


def _body(eps_ref, a_ref, x_ref, w1_ref, b1_ref, g1_ref, be1_ref,
          w2_ref, b2_ref, g2_ref, be2_ref, v1_ref, c1_ref, v2_ref, c2_ref,
          out_ref, q_ref, g_ref):
    n = a_ref.shape[1]
    d = x_ref.shape[2]
    nt = n // TK
    f32 = jnp.float32
    a = a_ref[0]
    x = x_ref[0]
    g_ref[:, :] = jnp.dot(x, v1_ref[:, :], preferred_element_type=f32) \
        + 0.5 * c1_ref[:, :]
    G = g_ref[:, :]
    c2 = c2_ref[:, :]
    v2 = v2_ref[:, :]
    rowi = jax.lax.broadcasted_iota(jnp.int32, (n, n), 0)
    coli = jax.lax.broadcasted_iota(jnp.int32, (n, n), 1)
    atl = jnp.where(rowi > coli, a, 0.0)
    c2t = c2.reshape(-1, 1)
    for t in range(nt):
        rmax = (t + 1) * TK
        gk = G[t * TK:(t + 1) * TK, :]
        h = gk[:, None, :] + G[:rmax][None, :, :]
        h = jnp.maximum(h, 0.0).reshape(TK * rmax, -1)
        pt = jax.lax.dot_general(
            v2, h, dimension_numbers=(((0,), (1,)), ((), ())),
            preferred_element_type=f32) + c2t
        pt = jnp.maximum(pt, 0.0)
        for kk in range(TK):
            k = t * TK + kk
            q_ref[k * d:(k + 1) * d, :rmax] = (
                atl[k:k + 1, :rmax] * pt[:, kk * rmax:(kk + 1) * rmax])
    accT = jnp.zeros((d, n), dtype=f32)
    for t in range(nt):
        rmax = (t + 1) * TK
        yt = jax.lax.dot_general(
            q_ref[t * TK * d:(t + 1) * TK * d, :rmax], a[:, :rmax],
            dimension_numbers=(((1,), (1,)), ((), ())),
            preferred_element_type=f32)
        for kk in range(TK):
            k = t * TK + kk
            accT = accT + a[k:k + 1, :] * yt[kk * d:(kk + 1) * d, :]
    acc = accT.T
    xc = (1.0 + eps_ref[0, 0]) * x + jnp.dot(a, x, preferred_element_type=f32) \
        + acc
    inv = 1.0 / math.sqrt(1.0 + 1e-5)
    h1 = jnp.maximum(
        jnp.dot(xc, w1_ref[:, :], preferred_element_type=f32) + b1_ref[:, :], 0.0)
    h1 = h1 * (inv * g1_ref[:, :]) + be1_ref[:, :]
    h2 = jnp.maximum(
        jnp.dot(h1, w2_ref[:, :], preferred_element_type=f32) + b2_ref[:, :], 0.0)
    out_ref[0] = h2 * (inv * g2_ref[:, :]) + be2_ref[:, :]


def kernel(A, X, eps, W1, b1, g1, be1, W2, b2, g2, be2, V1, c1, V2, c2):
    b, n = A.shape[0], A.shape[1]
    d_in, d_h = W1.shape
    fixed = lambda *zeros: (lambda i: zeros)
    out = pl.pallas_call(
        _body,
        grid=(b,),
        in_specs=[
            pl.BlockSpec((1, 1), fixed(0, 0), memory_space=pltpu.SMEM),
            pl.BlockSpec((1, n, n), lambda i: (i, 0, 0)),
            pl.BlockSpec((1, n, d_in), lambda i: (i, 0, 0)),
            pl.BlockSpec((d_in, d_h), fixed(0, 0)),
            pl.BlockSpec((1, d_h), fixed(0, 0)),
            pl.BlockSpec((n, 1), fixed(0, 0)),
            pl.BlockSpec((n, 1), fixed(0, 0)),
            pl.BlockSpec((d_h, d_h), fixed(0, 0)),
            pl.BlockSpec((1, d_h), fixed(0, 0)),
            pl.BlockSpec((n, 1), fixed(0, 0)),
            pl.BlockSpec((n, 1), fixed(0, 0)),
            pl.BlockSpec((d_in, d_h), fixed(0, 0)),
            pl.BlockSpec((1, d_h), fixed(0, 0)),
            pl.BlockSpec((d_h, d_in), fixed(0, 0)),
            pl.BlockSpec((1, d_in), fixed(0, 0)),
        ],
        out_specs=pl.BlockSpec((1, n, d_h), lambda i: (i, 0, 0)),
        out_shape=jax.ShapeDtypeStruct((b, n, d_h), jnp.float32),
        scratch_shapes=[
            pltpu.VMEM((n * d_in, n), jnp.float32),
            pltpu.VMEM((n, d_h), jnp.float32),
        ],
        compiler_params=pltpu.CompilerParams(
            dimension_semantics=("parallel",),
        ),
    )(
        eps.reshape(1, 1), A, X, W1, b1.reshape(1, d_h), g1.reshape(n, 1),
        be1.reshape(n, 1), W2, b2.reshape(1, d_h), g2.reshape(n, 1),
        be2.reshape(n, 1), V1, c1.reshape(1, d_h), V2, c2.reshape(1, d_in),
    )
    return out
